# baseline (speedup 1.0000x reference)
import jax
import jax.numpy as jnp
from jax.experimental import pallas as pl


def _matmul_kernel(a_ref, b_ref, o_ref):
    o_ref[...] = jnp.dot(a_ref[...], b_ref[...],
                         preferred_element_type=jnp.float32)


def kernel(x, edge_index, K, W):
    N = x.shape[0]
    src = edge_index[0]
    dst = edge_index[1]
    x_src = jnp.take(x, src, axis=0)
    v0 = K[0][dst, src]
    v1 = K[1][dst, src]
    agg0 = jax.ops.segment_sum(v0[:, None] * x_src, dst, num_segments=N)
    agg1 = jax.ops.segment_sum(v1[:, None] * x_src, dst, num_segments=N)
    out = jnp.concatenate([agg0, agg1], axis=1) + jnp.tile(x, (1, 2))
    return pl.pallas_call(
        _matmul_kernel,
        out_shape=jax.ShapeDtypeStruct((N, W.shape[0]), jnp.float32),
    )(out, W.T)


# speedup vs baseline: 3.1593x; 3.1593x over previous
import functools

import jax
import jax.numpy as jnp
from jax import lax
from jax.experimental import pallas as pl
from jax.experimental.pallas import tpu as pltpu
from jax.experimental.pallas import tpu_sc as plsc

_N = 4096
_D = 128
_E = 131072
_NC = 2
_NS = 16
_NW = _NC * _NS
_EPW = _E // _NW
_B = 128
_NB = _EPW // _B
_RPT = _N // _NS


def _y_kernel(x_ref, wt_ref, o_ref):
    o_ref[...] = jnp.dot(x_ref[...], wt_ref[...],
                         preferred_element_type=jnp.float32)


def _combine_kernel(acc_ref, y_ref, o_ref):
    o_ref[...] = (acc_ref[0] + acc_ref[1]
                  + y_ref[:, :_D] + y_ref[:, _D:])


def _sc_body(src_hbm, dst_hbm, kflat_hbm, y01_hbm, out_hbm,
             src_v, dst_v, flat0_v, flat1_v, v0_v, v1_v, yrow_v, msg_v,
             acc_sh, sem0, sem1, sem2):
    c = lax.axis_index("c")
    s = lax.axis_index("s")
    wid = s * _NC + c

    zero16 = jnp.zeros((16,), jnp.float32)

    def _zrow(i, carry):
        for j in range(_D // 16):
            msg_v[i, pl.ds(j * 16, 16)] = zero16
        return carry

    lax.fori_loop(0, _B, _zrow, 0)
    for r in range(_RPT // _B):
        pltpu.sync_copy(msg_v, acc_sh.at[pl.ds(s * _RPT + r * _B, _B)])
    plsc.subcore_barrier()

    def _batch(b, carry):
        base = wid * _EPW + b * _B
        pltpu.sync_copy(src_hbm.at[pl.ds(base, _B)], src_v)
        pltpu.sync_copy(dst_hbm.at[pl.ds(base, _B)], dst_v)
        for j in range(_B // 16):
            sl = pl.ds(j * 16, 16)
            f0 = dst_v[sl] * _N + src_v[sl]
            flat0_v[sl] = f0
            flat1_v[sl] = f0 + _N * _N
        cp0 = pltpu.async_copy(kflat_hbm.at[flat0_v], v0_v, sem0)
        cp1 = pltpu.async_copy(kflat_hbm.at[flat1_v], v1_v, sem1)
        cpy = pltpu.async_copy(y01_hbm.at[src_v], yrow_v, sem2)
        cp0.wait()
        cp1.wait()
        cpy.wait()

        for cc in range(_B // 16):
            v0c = v0_v[pl.ds(cc * 16, 16)]
            v1c = v1_v[pl.ds(cc * 16, 16)]

            def _edge(l, carry2, v0c=v0c, v1c=v1c, cc=cc):
                i = cc * 16 + l
                lsplat = jnp.full((16,), l, jnp.int32)
                v0s = v0c.at[lsplat].get(mode="promise_in_bounds")
                v1s = v1c.at[lsplat].get(mode="promise_in_bounds")
                for j in range(_D // 16):
                    sl = pl.ds(j * 16, 16)
                    msg_v[i, sl] = (v0s * yrow_v[i, sl]
                                    + v1s * yrow_v[i, pl.ds(_D + j * 16, 16)])
                return carry2

            lax.fori_loop(0, 16, _edge, 0)
        pltpu.sync_copy(msg_v, acc_sh.at[dst_v], add=True)
        return carry

    lax.fori_loop(0, _NB, _batch, 0)
    plsc.subcore_barrier()

    for r in range(_RPT // _B):
        row0 = s * _RPT + r * _B
        pltpu.sync_copy(acc_sh.at[pl.ds(row0, _B)], msg_v)
        pltpu.sync_copy(msg_v, out_hbm.at[c, pl.ds(row0, _B)])


_sc_call = functools.partial(
    pl.kernel,
    mesh=plsc.VectorSubcoreMesh(core_axis_name="c", subcore_axis_name="s"),
    out_type=jax.ShapeDtypeStruct((_NC, _N, _D), jnp.float32),
    scratch_types=[
        pltpu.VMEM((_B,), jnp.int32),
        pltpu.VMEM((_B,), jnp.int32),
        pltpu.VMEM((_B,), jnp.int32),
        pltpu.VMEM((_B,), jnp.int32),
        pltpu.VMEM((_B,), jnp.float32),
        pltpu.VMEM((_B,), jnp.float32),
        pltpu.VMEM((_B, 2 * _D), jnp.float32),
        pltpu.VMEM((_B, _D), jnp.float32),
        pltpu.VMEM_SHARED((_N, _D), jnp.float32),
        pltpu.SemaphoreType.DMA,
        pltpu.SemaphoreType.DMA,
        pltpu.SemaphoreType.DMA,
    ],
)(_sc_body)


def kernel(x, edge_index, K, W):
    src = edge_index[0]
    dst = edge_index[1]
    kflat = K.reshape(2 * _N * _N)
    wstack = jnp.concatenate([W[:, :_D].T, W[:, _D:].T], axis=1)

    y01 = pl.pallas_call(
        _y_kernel,
        out_shape=jax.ShapeDtypeStruct((_N, 2 * _D), jnp.float32),
    )(x, wstack)

    acc = _sc_call(src, dst, kflat, y01)

    return pl.pallas_call(
        _combine_kernel,
        out_shape=jax.ShapeDtypeStruct((_N, _D), jnp.float32),
    )(acc, y01)


# speedup vs baseline: 4.1689x; 1.3196x over previous
import functools

import jax
import jax.numpy as jnp
from jax import lax
from jax.experimental import pallas as pl
from jax.experimental.pallas import tpu as pltpu
from jax.experimental.pallas import tpu_sc as plsc

_N = 4096
_D = 128
_E = 131072
_NC = 2
_NS = 16
_NW = _NC * _NS
_EPW = _E // _NW
_B = 64
_NB = _EPW // _B
_RPT = _N // _NS


def _y_kernel(x_ref, wt_ref, o_ref):
    o_ref[...] = jnp.dot(x_ref[...], wt_ref[...],
                         preferred_element_type=jnp.float32)


def _combine_kernel(acc_ref, y_ref, o_ref):
    o_ref[...] = (acc_ref[0] + acc_ref[1]
                  + y_ref[:, :_D] + y_ref[:, _D:])


def _sc_body(src_hbm, dst_hbm, kflat_hbm, y01_hbm, out_hbm,
             src_all, dst_all, flat0_all, flat1_all, dstsc,
             v0_v, v1_v, yrow_v, msg_v, acc_sh,
             sem_i, sem_g0, sem_g1, sem_s0, sem_s1):
    c = lax.axis_index("c")
    s = lax.axis_index("s")
    wid = s * _NC + c
    sem_g = [sem_g0, sem_g1]
    sem_s = [sem_s0, sem_s1]

    cpi0 = pltpu.async_copy(src_hbm.at[pl.ds(wid * _EPW, _EPW)], src_all,
                            sem_i)
    cpi1 = pltpu.async_copy(dst_hbm.at[pl.ds(wid * _EPW, _EPW)], dst_all,
                            sem_i)

    zero16 = jnp.zeros((16,), jnp.float32)

    def _zrow(i, carry):
        for j in range(_D // 16):
            msg_v[0, i, pl.ds(j * 16, 16)] = zero16
        return carry

    lax.fori_loop(0, _B, _zrow, 0)
    for r in range(_RPT // _B):
        pltpu.sync_copy(msg_v.at[0],
                        acc_sh.at[pl.ds(s * _RPT + r * _B, _B)])

    cpi0.wait()
    cpi1.wait()

    def _flp(t, carry):
        sl = pl.ds(t * 16, 16)
        f0 = dst_all[sl] * _N + src_all[sl]
        flat0_all[sl] = f0
        flat1_all[sl] = f0 + _N * _N
        return carry

    lax.fori_loop(0, _EPW // 16, _flp, 0)

    def _dsts(b, carry):
        for j in range(_B // 16):
            dstsc[b, pl.ds(j * 16, 16)] = dst_all[pl.ds(b * _B + j * 16, 16)]
        return carry

    lax.fori_loop(0, _NB, _dsts, 0)
    plsc.subcore_barrier()

    def _start_g(i, p):
        sl = pl.ds(i * _B, _B)
        pltpu.async_copy(kflat_hbm.at[flat0_all.at[sl]], v0_v.at[p],
                         sem_g[p])
        pltpu.async_copy(kflat_hbm.at[flat1_all.at[sl]], v1_v.at[p],
                         sem_g[p])
        pltpu.async_copy(y01_hbm.at[src_all.at[sl]], yrow_v.at[p], sem_g[p])

    def _wait_g(p):
        sl = pl.ds(0, _B)
        pltpu.make_async_copy(kflat_hbm.at[flat0_all.at[sl]], v0_v.at[p],
                              sem_g[p]).wait()
        pltpu.make_async_copy(kflat_hbm.at[flat1_all.at[sl]], v1_v.at[p],
                              sem_g[p]).wait()
        pltpu.make_async_copy(y01_hbm.at[src_all.at[sl]], yrow_v.at[p],
                              sem_g[p]).wait()

    def _start_s(i, p):
        pltpu.async_copy(msg_v.at[p], acc_sh.at[dstsc.at[i]], sem_s[p],
                         add=True)

    def _wait_s(p):
        pltpu.make_async_copy(msg_v.at[p], acc_sh.at[dstsc.at[0]],
                              sem_s[p]).wait()

    def _compute(p):
        for cc in range(_B // 16):
            v0c = v0_v[p, pl.ds(cc * 16, 16)]
            v1c = v1_v[p, pl.ds(cc * 16, 16)]

            def _edge(l, carry2, v0c=v0c, v1c=v1c, cc=cc):
                i = cc * 16 + l
                lsplat = jnp.full((16,), l, jnp.int32)
                v0s = v0c.at[lsplat].get(mode="promise_in_bounds")
                v1s = v1c.at[lsplat].get(mode="promise_in_bounds")
                for j in range(_D // 16):
                    sl = pl.ds(j * 16, 16)
                    msg_v[p, i, sl] = (
                        v0s * yrow_v[p, i, sl]
                        + v1s * yrow_v[p, i, pl.ds(_D + j * 16, 16)])
                return carry2

            lax.fori_loop(0, 16, _edge, 0)

    def _step(i, p):
        @pl.when(i + 1 < _NB)
        def _():
            _start_g(i + 1, 1 - p)

        _wait_g(p)

        @pl.when(i >= 2)
        def _():
            _wait_s(p)

        _compute(p)
        _start_s(i, p)

    _start_g(0, 0)

    def _pair(t, carry):
        i0 = 2 * t
        _step(i0, 0)
        _step(i0 + 1, 1)
        return carry

    lax.fori_loop(0, _NB // 2, _pair, 0)
    _wait_s(0)
    _wait_s(1)
    plsc.subcore_barrier()

    for r in range(_RPT // _B):
        row0 = s * _RPT + r * _B
        pltpu.sync_copy(acc_sh.at[pl.ds(row0, _B)], msg_v.at[0])
        pltpu.sync_copy(msg_v.at[0], out_hbm.at[c, pl.ds(row0, _B)])


_sc_call = functools.partial(
    pl.kernel,
    mesh=plsc.VectorSubcoreMesh(core_axis_name="c", subcore_axis_name="s"),
    out_type=jax.ShapeDtypeStruct((_NC, _N, _D), jnp.float32),
    scratch_types=[
        pltpu.VMEM((_EPW,), jnp.int32),
        pltpu.VMEM((_EPW,), jnp.int32),
        pltpu.VMEM((_EPW,), jnp.int32),
        pltpu.VMEM((_EPW,), jnp.int32),
        pltpu.VMEM((_NB, _B), jnp.int32),
        pltpu.VMEM((2, _B), jnp.float32),
        pltpu.VMEM((2, _B), jnp.float32),
        pltpu.VMEM((2, _B, 2 * _D), jnp.float32),
        pltpu.VMEM((2, _B, _D), jnp.float32),
        pltpu.VMEM_SHARED((_N, _D), jnp.float32),
        pltpu.SemaphoreType.DMA,
        pltpu.SemaphoreType.DMA,
        pltpu.SemaphoreType.DMA,
        pltpu.SemaphoreType.DMA,
        pltpu.SemaphoreType.DMA,
    ],
)(_sc_body)


def kernel(x, edge_index, K, W):
    src = edge_index[0]
    dst = edge_index[1]
    kflat = K.reshape(2 * _N * _N)
    wstack = jnp.concatenate([W[:, :_D].T, W[:, _D:].T], axis=1)

    y01 = pl.pallas_call(
        _y_kernel,
        out_shape=jax.ShapeDtypeStruct((_N, 2 * _D), jnp.float32),
    )(x, wstack)

    acc = _sc_call(src, dst, kflat, y01)

    return pl.pallas_call(
        _combine_kernel,
        out_shape=jax.ShapeDtypeStruct((_N, _D), jnp.float32),
    )(acc, y01)
